# baseline (speedup 1.0000x reference)
import jax
import jax.numpy as jnp
from jax.experimental import pallas as pl
from jax.experimental.pallas import tpu as pltpu

N3 = 300000
N = N3 // 3
D = 128
B = 10000
NZ = N // B


def _body(z_ref, zp_ref, zn_ref, wz_ref, wp_ref, wn_ref, o_ref,
          s_ref, sc_ref, ls_ref, vacc_ref):
    g = pl.program_id(0)

    @pl.when(g == 0)
    def _init():
        s_ref[...] = jnp.zeros_like(s_ref)
        vacc_ref[...] = jnp.zeros_like(vacc_ref)
        sc_ref[0] = 0.0
        sc_ref[1] = 0.0
        sc_ref[2] = 0.0
        sc_ref[3] = 0.0

    wz = wz_ref[0]
    wp = wp_ref[0]
    wn = wn_ref[0]

    ones_row = jnp.ones((1, D), jnp.bfloat16)
    negones_row = jnp.full((1, D), -1.0, jnp.bfloat16)

    def lanered(v, e):
        return jax.lax.dot_general(
            v, e, (((1,), (1,)), ((), ())),
            preferred_element_type=jnp.float32)

    def colsum(w, x):
        return jax.lax.dot_general(
            w, x, (((1,), (0,)), ((), ())),
            preferred_element_type=jnp.float32)

    @pl.when(g < NZ)
    def _phase_a():
        zb = z_ref[...].astype(jnp.bfloat16)
        zpb = zp_ref[...].astype(jnp.bfloat16)
        znb = zn_ref[...].astype(jnp.bfloat16)
        ls_ref[pl.ds(g, 1), :] = lanered(ones_row, zb * zpb)
        ls_ref[pl.ds(NZ + g, 1), :] = lanered(negones_row, zb * znb)
        s_ref[...] += (colsum(wz.astype(jnp.bfloat16), zb)
                       + colsum(wp.astype(jnp.bfloat16), zpb)
                       + colsum(wn.astype(jnp.bfloat16), znb))
        vacc_ref[...] += wz + wp + wn

    @pl.when(g == NZ)
    def _count():
        sc_ref[2] = jnp.sum(vacc_ref[...])

    @pl.when(g >= NZ)
    def _phase_b():
        mu = s_ref[...] / jnp.maximum(sc_ref[2], 1.0)
        mub = mu.astype(jnp.bfloat16)

        def contrib(x_ref, w):
            xm = x_ref[...].astype(jnp.bfloat16) - mub
            d2 = lanered(ones_row, xm * xm)
            return jnp.sum(jnp.sqrt(w * d2))

        sc_ref[3] += (contrib(z_ref, wz) + contrib(zp_ref, wp)
                      + contrib(zn_ref, wn))

    @pl.when(g == 2 * NZ - 1)
    def _fin():
        v = ls_ref[...]
        sc_ref[0] = jnp.sum(jnp.minimum(v, 0.0)
                            - jnp.log1p(jnp.exp(-jnp.abs(v))))
        sig = 1.0 / (1.0 + jnp.exp(-sc_ref[3]))
        total = -sc_ref[0] / N + sig - 0.5
        o_ref[...] = jnp.full((1, 1), total, dtype=jnp.float32)


def kernel(out, mask):
    w = mask.astype(jnp.float32).reshape(3 * NZ, 1, B)

    def omap(t):
        return lambda g: (t * NZ + g % NZ, 0)

    def wmap(t):
        return lambda g: (t * NZ + g % NZ, 0, 0)

    res = pl.pallas_call(
        _body,
        grid=(2 * NZ,),
        in_specs=[
            pl.BlockSpec((B, D), omap(0)),
            pl.BlockSpec((B, D), omap(1)),
            pl.BlockSpec((B, D), omap(2)),
            pl.BlockSpec((1, 1, B), wmap(0)),
            pl.BlockSpec((1, 1, B), wmap(1)),
            pl.BlockSpec((1, 1, B), wmap(2)),
        ],
        out_specs=pl.BlockSpec((1, 1), lambda g: (0, 0)),
        out_shape=jax.ShapeDtypeStruct((1, 1), jnp.float32),
        scratch_shapes=[
            pltpu.VMEM((1, D), jnp.float32),
            pltpu.SMEM((4,), jnp.float32),
            pltpu.VMEM((2 * NZ, B), jnp.float32),
            pltpu.VMEM((1, B), jnp.float32),
        ],
        compiler_params=pltpu.CompilerParams(
            dimension_semantics=("arbitrary",),
        ),
    )(out, out, out, w, w, w)
    return res[0, 0]


# speedup vs baseline: 1.1024x; 1.1024x over previous
import jax
import jax.numpy as jnp
from jax.experimental import pallas as pl
from jax.experimental.pallas import tpu as pltpu

N3 = 300000
N = N3 // 3
D = 128
B = 5000
NZ = N // B


def _body(z_ref, zp_ref, zn_ref, wz_ref, wp_ref, wn_ref, o_ref,
          s_ref, sc_ref, cz_ref, cp_ref, cn_ref):
    g = pl.program_id(0)

    @pl.when(g == 0)
    def _init():
        s_ref[...] = jnp.zeros_like(s_ref)
        sc_ref[0] = 0.0
        sc_ref[1] = 0.0
        sc_ref[2] = 0.0
        sc_ref[3] = 0.0

    wz = wz_ref[0]
    wp = wp_ref[0]
    wn = wn_ref[0]

    ones_row = jnp.ones((1, D), jnp.bfloat16)

    def lanered(v, e):
        return jax.lax.dot_general(
            v, e, (((1,), (1,)), ((), ())),
            preferred_element_type=jnp.float32)

    def colsum(w, x):
        return jax.lax.dot_general(
            w, x, (((1,), (0,)), ((), ())),
            preferred_element_type=jnp.float32)

    def logsig_sum(x):
        return jnp.sum(jnp.minimum(x, 0.0) - jnp.log1p(jnp.exp(-jnp.abs(x))))

    @pl.when(g < NZ)
    def _phase_a():
        z = z_ref[...]
        zp = zp_ref[...]
        zn = zn_ref[...]
        zb = z.astype(jnp.bfloat16)
        zpb = zp.astype(jnp.bfloat16)
        znb = zn.astype(jnp.bfloat16)
        dp = lanered(ones_row, zb * zpb)
        dn = lanered(ones_row, zb * znb)
        sc_ref[0] += logsig_sum(dp)
        sc_ref[1] += logsig_sum(-dn)
        s_ref[...] += (colsum(wz.astype(jnp.bfloat16), zb)
                       + colsum(wp.astype(jnp.bfloat16), zpb)
                       + colsum(wn.astype(jnp.bfloat16), znb))
        sc_ref[2] += jnp.sum(wz) + jnp.sum(wp) + jnp.sum(wn)
        cz_ref[g] = z.astype(jnp.float8_e4m3fn)
        cp_ref[g] = zp.astype(jnp.float8_e4m3fn)
        cn_ref[g] = zn.astype(jnp.float8_e4m3fn)

    @pl.when(g >= NZ)
    def _phase_b():
        j = g - NZ
        mu = s_ref[...] / jnp.maximum(sc_ref[2], 1.0)
        mub = mu.astype(jnp.bfloat16)

        def contrib(c_ref, w):
            xm = c_ref[j].astype(jnp.bfloat16) - mub
            d2 = lanered(ones_row, xm * xm)
            return jnp.sum(jnp.sqrt(w * d2))

        sc_ref[3] += (contrib(cz_ref, wz) + contrib(cp_ref, wp)
                      + contrib(cn_ref, wn))

    @pl.when(g == 2 * NZ - 1)
    def _fin():
        sig = 1.0 / (1.0 + jnp.exp(-sc_ref[3]))
        total = -(sc_ref[0] + sc_ref[1]) / N + sig - 0.5
        o_ref[...] = jnp.full((1, 1), total, dtype=jnp.float32)


def kernel(out, mask):
    w = mask.astype(jnp.float32).reshape(3 * NZ, 1, B)

    def omap(t):
        def f(g):
            idx = jnp.minimum(g, NZ - 1)
            return (t * NZ + idx, 0)
        return f

    def wmap(t):
        return lambda g: (t * NZ + g % NZ, 0, 0)

    res = pl.pallas_call(
        _body,
        grid=(2 * NZ,),
        in_specs=[
            pl.BlockSpec((B, D), omap(0)),
            pl.BlockSpec((B, D), omap(1)),
            pl.BlockSpec((B, D), omap(2)),
            pl.BlockSpec((1, 1, B), wmap(0)),
            pl.BlockSpec((1, 1, B), wmap(1)),
            pl.BlockSpec((1, 1, B), wmap(2)),
        ],
        out_specs=pl.BlockSpec((1, 1), lambda g: (0, 0)),
        out_shape=jax.ShapeDtypeStruct((1, 1), jnp.float32),
        scratch_shapes=[
            pltpu.VMEM((1, D), jnp.float32),
            pltpu.SMEM((4,), jnp.float32),
            pltpu.VMEM((NZ, B, D), jnp.float8_e4m3fn),
            pltpu.VMEM((NZ, B, D), jnp.float8_e4m3fn),
            pltpu.VMEM((NZ, B, D), jnp.float8_e4m3fn),
        ],
        compiler_params=pltpu.CompilerParams(
            dimension_semantics=("arbitrary",),
        ),
    )(out, out, out, w, w, w)
    return res[0, 0]


# speedup vs baseline: 1.1444x; 1.0380x over previous
import jax
import jax.numpy as jnp
from jax.experimental import pallas as pl
from jax.experimental.pallas import tpu as pltpu

N3 = 300000
N = N3 // 3
D = 128
B = 4000
NZ = N // B


def _body(z_ref, zp_ref, zn_ref, wz_ref, wp_ref, wn_ref, o_ref,
          s_ref, sc_ref, cz_ref, cp_ref, cn_ref, q_ref):
    g = pl.program_id(0)

    @pl.when(g == 0)
    def _init():
        s_ref[...] = jnp.zeros_like(s_ref)
        sc_ref[0] = 0.0
        sc_ref[1] = 0.0
        sc_ref[2] = 0.0
        sc_ref[3] = 0.0

    wz = wz_ref[0]
    wp = wp_ref[0]
    wn = wn_ref[0]

    ones_row = jnp.ones((1, D), jnp.bfloat16)

    def lanered(v, e):
        return jax.lax.dot_general(
            v, e, (((1,), (1,)), ((), ())),
            preferred_element_type=jnp.float32)

    def colsum(w, x):
        return jax.lax.dot_general(
            w, x, (((1,), (0,)), ((), ())),
            preferred_element_type=jnp.float32)

    def logsig_sum(x):
        return jnp.sum(jnp.minimum(x, 0.0) - jnp.log1p(jnp.exp(-jnp.abs(x))))

    @pl.when(g < NZ)
    def _phase_a():
        z = z_ref[...]
        zp = zp_ref[...]
        zn = zn_ref[...]
        zb = z.astype(jnp.bfloat16)
        zpb = zp.astype(jnp.bfloat16)
        znb = zn.astype(jnp.bfloat16)
        z8 = z.astype(jnp.float8_e4m3fn)
        zp8 = zp.astype(jnp.float8_e4m3fn)
        zn8 = zn.astype(jnp.float8_e4m3fn)
        dp = lanered(ones_row, zb * zpb)
        dn = lanered(ones_row, zb * znb)
        sc_ref[0] += logsig_sum(dp)
        sc_ref[1] += logsig_sum(-dn)
        w8 = jnp.float8_e4m3fn
        s_ref[...] += (colsum(wz.astype(w8), z8)
                       + colsum(wp.astype(w8), zp8)
                       + colsum(wn.astype(w8), zn8))
        sc_ref[2] += jnp.sum(wz) + jnp.sum(wp) + jnp.sum(wn)
        q_ref[pl.ds(g, 1), :] = lanered(ones_row, zb * zb)
        q_ref[pl.ds(NZ + g, 1), :] = lanered(ones_row, zpb * zpb)
        q_ref[pl.ds(2 * NZ + g, 1), :] = lanered(ones_row, znb * znb)
        cz_ref[g] = z8
        cp_ref[g] = zp8
        cn_ref[g] = zn8

    @pl.when(g >= NZ)
    def _phase_b():
        j = g - NZ
        mu = s_ref[...] / jnp.maximum(sc_ref[2], 1.0)
        m = jnp.sum(mu * mu)
        mu8 = (mu * -128.0).astype(jnp.float8_e4m3fn)

        def contrib(c_ref, t, w):
            p = lanered(mu8, c_ref[j])
            d2 = q_ref[pl.ds(t * NZ + j, 1), :] + p * (1.0 / 64.0) + m
            return jnp.sum(jnp.sqrt(jnp.maximum(w * d2, 0.0)))

        sc_ref[3] += (contrib(cz_ref, 0, wz) + contrib(cp_ref, 1, wp)
                      + contrib(cn_ref, 2, wn))

    @pl.when(g == 2 * NZ - 1)
    def _fin():
        sig = 1.0 / (1.0 + jnp.exp(-sc_ref[3]))
        total = -(sc_ref[0] + sc_ref[1]) / N + sig - 0.5
        o_ref[...] = jnp.full((1, 1), total, dtype=jnp.float32)


def kernel(out, mask):
    w = mask.astype(jnp.float32).reshape(3 * NZ, 1, B)

    def omap(t):
        def f(g):
            idx = jnp.minimum(g, NZ - 1)
            return (t * NZ + idx, 0)
        return f

    def wmap(t):
        return lambda g: (t * NZ + g % NZ, 0, 0)

    res = pl.pallas_call(
        _body,
        grid=(2 * NZ,),
        in_specs=[
            pl.BlockSpec((B, D), omap(0)),
            pl.BlockSpec((B, D), omap(1)),
            pl.BlockSpec((B, D), omap(2)),
            pl.BlockSpec((1, 1, B), wmap(0)),
            pl.BlockSpec((1, 1, B), wmap(1)),
            pl.BlockSpec((1, 1, B), wmap(2)),
        ],
        out_specs=pl.BlockSpec((1, 1), lambda g: (0, 0)),
        out_shape=jax.ShapeDtypeStruct((1, 1), jnp.float32),
        scratch_shapes=[
            pltpu.VMEM((1, D), jnp.float32),
            pltpu.SMEM((4,), jnp.float32),
            pltpu.VMEM((NZ, B, D), jnp.float8_e4m3fn),
            pltpu.VMEM((NZ, B, D), jnp.float8_e4m3fn),
            pltpu.VMEM((NZ, B, D), jnp.float8_e4m3fn),
            pltpu.VMEM((3 * NZ, B), jnp.float32),
        ],
        compiler_params=pltpu.CompilerParams(
            dimension_semantics=("arbitrary",),
        ),
    )(out, out, out, w, w, w)
    return res[0, 0]


# speedup vs baseline: 1.2712x; 1.1109x over previous
import jax
import jax.numpy as jnp
from jax.experimental import pallas as pl
from jax.experimental.pallas import tpu as pltpu

N3 = 300000
N = N3 // 3
D = 128
B = 4000
NZ = N // B
F8 = jnp.float8_e4m3fn


def _body(z_ref, zp_ref, zn_ref, wz_ref, wp_ref, wn_ref, o_ref,
          s_ref, sc_ref, cz_ref, cp_ref, cn_ref, q_ref, ls_ref, wv_ref):
    g = pl.program_id(0)

    @pl.when(g == 0)
    def _init():
        s_ref[...] = jnp.zeros_like(s_ref)
        sc_ref[2] = 0.0

    ones_row = jnp.ones((1, D), jnp.bfloat16)
    negones_row = jnp.full((1, D), -1.0, jnp.bfloat16)

    def lanered(v, e):
        return jax.lax.dot_general(
            v, e, (((1,), (1,)), ((), ())),
            preferred_element_type=jnp.float32)

    def colsum(w, x):
        return jax.lax.dot_general(
            w, x, (((1,), (0,)), ((), ())),
            preferred_element_type=jnp.float32)

    @pl.when(g < NZ)
    def _phase_a():
        wz = wz_ref[0]
        wp = wp_ref[0]
        wn = wn_ref[0]
        zb = z_ref[...].astype(jnp.bfloat16)
        zpb = zp_ref[...].astype(jnp.bfloat16)
        znb = zn_ref[...].astype(jnp.bfloat16)
        z8 = zb.astype(F8)
        zp8 = zpb.astype(F8)
        zn8 = znb.astype(F8)
        ls_ref[pl.ds(g, 1), :] = lanered(ones_row, zb * zpb)
        ls_ref[pl.ds(NZ + g, 1), :] = lanered(negones_row, zb * znb)
        s_ref[...] += (colsum(wz.astype(F8), z8)
                       + colsum(wp.astype(F8), zp8)
                       + colsum(wn.astype(F8), zn8))
        sc_ref[2] += jnp.sum(wz) + jnp.sum(wp) + jnp.sum(wn)
        q_ref[pl.ds(g, 1), :] = lanered(ones_row, zb * zb)
        q_ref[pl.ds(NZ + g, 1), :] = lanered(ones_row, zpb * zpb)
        q_ref[pl.ds(2 * NZ + g, 1), :] = lanered(ones_row, znb * znb)
        wv_ref[pl.ds(g, 1), :] = wz
        wv_ref[pl.ds(NZ + g, 1), :] = wp
        wv_ref[pl.ds(2 * NZ + g, 1), :] = wn
        cz_ref[g] = z8
        cp_ref[g] = zp8
        cn_ref[g] = zn8

    @pl.when(g == NZ)
    def _finish():
        v = ls_ref[...]
        lssum = jnp.sum(jnp.minimum(v, 0.0)
                        - jnp.log1p(jnp.exp(-jnp.abs(v))))

        mu = s_ref[...] / jnp.maximum(sc_ref[2], 1.0)
        m = jnp.sum(mu * mu)
        mu8 = (mu * -128.0).astype(F8)

        coag = jnp.float32(0.0)
        for t, c_ref in enumerate((cz_ref, cp_ref, cn_ref)):
            for j in range(NZ):
                p = lanered(mu8, c_ref[j])
                d2 = q_ref[pl.ds(t * NZ + j, 1), :] + p * (1.0 / 64.0) + m
                w = wv_ref[pl.ds(t * NZ + j, 1), :]
                coag += jnp.sum(jnp.sqrt(jnp.maximum(w * d2, 0.0)))

        sig = 1.0 / (1.0 + jnp.exp(-coag))
        total = -lssum / N + sig - 0.5
        o_ref[...] = jnp.full((1, 1), total, dtype=jnp.float32)


def kernel(out, mask):
    w = mask.astype(jnp.float32).reshape(3 * NZ, 1, B)

    def omap(t):
        return lambda g: (t * NZ + jnp.minimum(g, NZ - 1), 0)

    def wmap(t):
        return lambda g: (t * NZ + jnp.minimum(g, NZ - 1), 0, 0)

    res = pl.pallas_call(
        _body,
        grid=(NZ + 1,),
        in_specs=[
            pl.BlockSpec((B, D), omap(0)),
            pl.BlockSpec((B, D), omap(1)),
            pl.BlockSpec((B, D), omap(2)),
            pl.BlockSpec((1, 1, B), wmap(0)),
            pl.BlockSpec((1, 1, B), wmap(1)),
            pl.BlockSpec((1, 1, B), wmap(2)),
        ],
        out_specs=pl.BlockSpec((1, 1), lambda g: (0, 0)),
        out_shape=jax.ShapeDtypeStruct((1, 1), jnp.float32),
        scratch_shapes=[
            pltpu.VMEM((1, D), jnp.float32),
            pltpu.SMEM((4,), jnp.float32),
            pltpu.VMEM((NZ, B, D), F8),
            pltpu.VMEM((NZ, B, D), F8),
            pltpu.VMEM((NZ, B, D), F8),
            pltpu.VMEM((3 * NZ, B), jnp.float32),
            pltpu.VMEM((2 * NZ, B), jnp.float32),
            pltpu.VMEM((3 * NZ, B), jnp.float32),
        ],
        compiler_params=pltpu.CompilerParams(
            dimension_semantics=("arbitrary",),
        ),
    )(out, out, out, w, w, w)
    return res[0, 0]


# speedup vs baseline: 1.2860x; 1.0116x over previous
import jax
import jax.numpy as jnp
from jax.experimental import pallas as pl
from jax.experimental.pallas import tpu as pltpu

N3 = 300000
N = N3 // 3
D = 128
B = 4000
NZ = N // B
F8 = jnp.float8_e4m3fn


def _body(z_ref, zp_ref, zn_ref, w_ref, o_ref,
          s_ref, sc_ref, cz_ref, cp_ref, cn_ref, q_ref, ls_ref, pv_ref):
    g = pl.program_id(0)

    @pl.when(g == 0)
    def _init():
        s_ref[...] = jnp.zeros_like(s_ref)

    ones_row = jnp.ones((1, D), jnp.bfloat16)
    negones_row = jnp.full((1, D), -1.0, jnp.bfloat16)

    def lanered(v, e):
        return jax.lax.dot_general(
            v, e, (((1,), (1,)), ((), ())),
            preferred_element_type=jnp.float32)

    def colsum(w, x):
        return jax.lax.dot_general(
            w, x, (((1,), (0,)), ((), ())),
            preferred_element_type=jnp.float32)

    @pl.when(g < NZ)
    def _phase_a():
        zb = z_ref[...].astype(jnp.bfloat16)
        zpb = zp_ref[...].astype(jnp.bfloat16)
        znb = zn_ref[...].astype(jnp.bfloat16)
        z8 = zb.astype(F8)
        zp8 = zpb.astype(F8)
        zn8 = znb.astype(F8)
        ls_ref[pl.ds(g, 1), :] = lanered(ones_row, zb * zpb)
        ls_ref[pl.ds(NZ + g, 1), :] = lanered(negones_row, zb * znb)
        s_ref[...] += (
            colsum(w_ref[pl.ds(g, 1), :].astype(F8), z8)
            + colsum(w_ref[pl.ds(NZ + g, 1), :].astype(F8), zp8)
            + colsum(w_ref[pl.ds(2 * NZ + g, 1), :].astype(F8), zn8))
        q_ref[pl.ds(g, 1), :] = lanered(ones_row, zb * zb)
        q_ref[pl.ds(NZ + g, 1), :] = lanered(ones_row, zpb * zpb)
        q_ref[pl.ds(2 * NZ + g, 1), :] = lanered(ones_row, znb * znb)
        cz_ref[g] = z8
        cp_ref[g] = zp8
        cn_ref[g] = zn8

    @pl.when(g == NZ)
    def _finish():
        v = ls_ref[...]
        lssum = jnp.sum(jnp.minimum(v, 0.0)
                        - jnp.log1p(jnp.exp(-jnp.abs(v))))

        wall = w_ref[...]
        cnt = jnp.maximum(jnp.sum(wall), 1.0)
        mu = s_ref[...] / cnt
        m = jnp.sum(mu * mu)
        mu8 = (mu * -128.0).astype(F8)

        for t, c_ref in enumerate((cz_ref, cp_ref, cn_ref)):
            for j in range(NZ):
                pv_ref[pl.ds(t * NZ + j, 1), :] = lanered(mu8, c_ref[j])

        d2 = q_ref[...] + pv_ref[...] * (1.0 / 64.0) + m
        coag = jnp.sum(jnp.sqrt(jnp.maximum(wall * d2, 0.0)))

        sig = 1.0 / (1.0 + jnp.exp(-coag))
        total = -lssum / N + sig - 0.5
        o_ref[...] = jnp.full((1, 1), total, dtype=jnp.float32)


def kernel(out, mask):
    w = mask.astype(jnp.float32).reshape(3 * NZ, B)

    def omap(t):
        return lambda g: (t * NZ + jnp.minimum(g, NZ - 1), 0)

    res = pl.pallas_call(
        _body,
        grid=(NZ + 1,),
        in_specs=[
            pl.BlockSpec((B, D), omap(0)),
            pl.BlockSpec((B, D), omap(1)),
            pl.BlockSpec((B, D), omap(2)),
            pl.BlockSpec((3 * NZ, B), lambda g: (0, 0)),
        ],
        out_specs=pl.BlockSpec((1, 1), lambda g: (0, 0)),
        out_shape=jax.ShapeDtypeStruct((1, 1), jnp.float32),
        scratch_shapes=[
            pltpu.VMEM((1, D), jnp.float32),
            pltpu.SMEM((4,), jnp.float32),
            pltpu.VMEM((NZ, B, D), F8),
            pltpu.VMEM((NZ, B, D), F8),
            pltpu.VMEM((NZ, B, D), F8),
            pltpu.VMEM((3 * NZ, B), jnp.float32),
            pltpu.VMEM((2 * NZ, B), jnp.float32),
            pltpu.VMEM((3 * NZ, B), jnp.float32),
        ],
        compiler_params=pltpu.CompilerParams(
            dimension_semantics=("arbitrary",),
        ),
    )(out, out, out, w)
    return res[0, 0]
